# Initial kernel scaffold; baseline (speedup 1.0000x reference)
import jax
import jax.numpy as jnp
from jax.experimental import pallas as pl


def kernel(span_vecs, span_mask, span_begin, span_end, gold_span_tensors, gold_spans_lengths, sequence_lengths):
    raise NotImplementedError("write your pallas kernel here")


# speedup vs baseline: 2.6156x; 2.6156x over previous
import functools

import jax
import jax.numpy as jnp
from jax import lax
from jax.experimental import pallas as pl
from jax.experimental.pallas import tpu as pltpu
from jax.experimental.pallas import tpu_sc as plsc

B, T, W, D, G = 8, 2048, 16, 128, 512
MAX_SPAN_LENGTH = 16
BIG = T * MAX_SPAN_LENGTH


def _sort_mask_body(lens_ref, gold_r_ref, gold_c_ref,
                    sorted_ref, reindex_ref, spanb_ref, spane_ref,
                    fb_ref, fe_ref, fs_ref, gidx_ref, sq_ref, tri_ref):
    b = pl.program_id(0)
    ln = lens_ref[b]

    gb_r = gold_r_ref[0, :, 0:1]
    ge_r = gold_r_ref[0, :, 1:2]
    gb_c = gold_c_ref[0, 0:1, :]
    ge_c = gold_c_ref[0, 1:2, :]
    pos_r = lax.broadcasted_iota(jnp.int32, (G, 1), 0)
    pos_c = lax.broadcasted_iota(jnp.int32, (1, G), 1)

    key_r = gb_r * MAX_SPAN_LENGTH + (ge_r - gb_r)
    key_c = gb_c * MAX_SPAN_LENGTH + (ge_c - gb_c)
    masked_r = jnp.where(pos_r < ln, key_r, BIG)
    masked_c = jnp.where(pos_c < ln, key_c, BIG)
    ck_r = masked_r * G + pos_r
    ck_c = masked_c * G + pos_c

    lt = (ck_c < ck_r).astype(jnp.int32)
    rank_r = jnp.sum(lt, axis=1, keepdims=True)
    eq = rank_r == pos_c
    pos_m = jnp.broadcast_to(pos_r, (G, G))
    val_m = jnp.broadcast_to(masked_r, (G, G))
    reindex = jnp.sum(jnp.where(eq, pos_m, 0), axis=0, keepdims=True)
    sortedm = jnp.sum(jnp.where(eq, val_m, 0), axis=0, keepdims=True)

    sidx = jnp.where(sortedm < BIG, sortedm, 0)
    sb = sidx >> 4
    se = sb + (sidx & (MAX_SPAN_LENGTH - 1))

    sorted_ref[0] = sidx
    reindex_ref[0] = reindex
    spanb_ref[0] = sb
    spane_ref[0] = se
    fb_ref[0] = sb.astype(jnp.float32)
    fe_ref[0] = se.astype(jnp.float32)
    fs_ref[0] = jnp.zeros((1, G), jnp.float32)
    gidx_ref[0] = sidx + b * BIG

    ii = lax.broadcasted_iota(jnp.int32, (G, G), 0)
    jj = lax.broadcasted_iota(jnp.int32, (G, G), 1)
    vm = (ii < ln) & (jj < ln)
    sq_ref[0] = jnp.where(vm, 1.0, 0.0)
    tri_ref[0] = jnp.where(vm & (jj <= ii), 1.0, 0.0)


def _sort_and_masks(lengths, gold):
    gold_c = jnp.transpose(gold, (0, 2, 1))
    vec = jax.ShapeDtypeStruct((B, 1, G), jnp.int32)
    vecf = jax.ShapeDtypeStruct((B, 1, G), jnp.float32)
    mask = jax.ShapeDtypeStruct((B, G, G), jnp.float32)
    vspec = pl.BlockSpec((1, 1, G), lambda b: (b, 0, 0))
    mspec = pl.BlockSpec((1, G, G), lambda b: (b, 0, 0))
    return pl.pallas_call(
        _sort_mask_body,
        grid=(B,),
        in_specs=[
            pl.BlockSpec(memory_space=pltpu.SMEM),
            pl.BlockSpec((1, G, 2), lambda b: (b, 0, 0)),
            pl.BlockSpec((1, 2, G), lambda b: (b, 0, 0)),
        ],
        out_specs=[vspec, vspec, vspec, vspec, vspec, vspec, vspec, vspec,
                   mspec, mspec],
        out_shape=[vec, vec, vec, vec, vecf, vecf, vecf, vec, mask, mask],
    )(lengths, gold, gold_c)


_NW = 32
_ROWS = B * G // _NW


@functools.cache
def _make_sc_gather():
    @functools.partial(
        pl.kernel,
        mesh=plsc.VectorSubcoreMesh(core_axis_name="c", subcore_axis_name="s"),
        out_type=jax.ShapeDtypeStruct((B * G, D), jnp.float32),
        scratch_types=[
            pltpu.VMEM((_ROWS,), jnp.int32),
            pltpu.VMEM((_ROWS, D), jnp.float32),
            pltpu.SemaphoreType.DMA,
        ],
    )
    def _sc_gather(table_hbm, idx_hbm, out_hbm, idx_v, rows_v, sem):
        wid = lax.axis_index("s") * 2 + lax.axis_index("c")
        base = wid * _ROWS
        pltpu.sync_copy(idx_hbm.at[pl.ds(base, _ROWS)], idx_v)
        pltpu.async_copy(table_hbm.at[idx_v], rows_v, sem).wait()
        pltpu.sync_copy(rows_v, out_hbm.at[pl.ds(base, _ROWS)])

    return _sc_gather


def kernel(span_vecs, span_mask, span_begin, span_end,
           gold_span_tensors, gold_spans_lengths, sequence_lengths):
    (sidx, reindex, sb, se, fb, fe, fs, gidx, sq, tri) = _sort_and_masks(
        gold_spans_lengths, gold_span_tensors)

    table = span_vecs.reshape(B * T * W, D)
    f_vecs = _make_sc_gather()(table, gidx.reshape(B * G)).reshape(B, G, D)

    return (f_vecs,
            fs.reshape(B, G, 1),
            fb.reshape(B, G, 1),
            fe.reshape(B, G, 1),
            sq,
            tri,
            sb.reshape(B, G),
            se.reshape(B, G),
            sidx.reshape(B, G),
            reindex.reshape(B, G))
